# baseline (speedup 1.0000x reference)
import functools

import jax
import jax.numpy as jnp
from jax import lax
from jax.experimental import pallas as pl
from jax.experimental.pallas import tpu as pltpu
from jax.experimental.pallas import tpu_sc as plsc

BATCH = 16384
EMBEDDING_DIM = 64
NUM_ROWS = 1_000_000

_INFO = plsc.get_sparse_core_info()
_NC = _INFO.num_cores
_NS = _INFO.num_subcores
_NW = _NC * _NS
_L = 16

_SLOT_W = 512
_NSLOT = (NUM_ROWS + _SLOT_W - 1) // _SLOT_W
_LAST_BASE = 999552
_ICHUNK = 1024
_NICHUNK = BATCH // _ICHUNK
_IVEC = _ICHUNK // _L
_NGROUP = 8


@functools.partial(
    pl.kernel,
    mesh=plsc.VectorSubcoreMesh(core_axis_name="c", subcore_axis_name="s"),
    out_type=jax.ShapeDtypeStruct((BATCH * EMBEDDING_DIM,), jnp.float32),
    scratch_types=[
        pltpu.VMEM((BATCH,), jnp.int32),
        pltpu.VMEM((BATCH,), jnp.int32),
        pltpu.VMEM((BATCH,), jnp.int32),
        pltpu.VMEM((2, EMBEDDING_DIM, _SLOT_W), jnp.float32),
        pltpu.VMEM((_L * EMBEDDING_DIM,), jnp.float32),
        pltpu.SemaphoreType.DMA,
        pltpu.SemaphoreType.DMA,
        pltpu.SemaphoreType.DMA,
    ],
    compiler_params=pltpu.CompilerParams(
        use_tc_tiling_on_sc=True,
        disable_bounds_checks=True,
        needs_layout_passes=False,
    ),
)
def _sc_gather(idx_hbm, table_t_hbm, out_hbm, idx_v, list_a, grp_p,
               block_v, stage_v, sem_b0, sem_b1, sem_out):
    wid = lax.axis_index("s") * _NC + lax.axis_index("c")
    s_lo = wid * _NSLOT // _NW
    s_hi = (wid + 1) * _NSLOT // _NW
    lo = s_lo * _SLOT_W
    width = jnp.minimum(s_hi * _SLOT_W, NUM_ROWS) - lo
    lanes = lax.iota(jnp.int32, _L)
    sems = [sem_b0, sem_b1]

    pltpu.sync_copy(idx_hbm, idx_v)

    def scan_strip(v, cnt):
        rel = idx_v[pl.ds(v * _L, _L)] - lo
        m = rel.astype(jnp.uint32) < width.astype(jnp.uint32)
        mi = m.astype(jnp.int32)
        pos = cnt + plsc.cumsum(mi) - mi
        packed = rel + ((v * _L + lanes) << 15)
        plsc.store_scatter(list_a, [pos], packed, mask=m)
        return cnt + plsc.all_reduce_population_count(m)[0]

    nhit = lax.fori_loop(0, BATCH // _L, scan_strip, jnp.int32(0))
    nhit_vecs = (nhit + _L - 1) // _L

    bounds = [jnp.int32(0)]

    def bucket(gi, gpos):
        def body(u, gpos):
            hp = list_a[pl.ds(u * _L, _L)]
            m = (((hp & 0x7FFF) >> 12) == gi) & (u * _L + lanes < nhit)
            mi = m.astype(jnp.int32)
            pos = gpos + plsc.cumsum(mi) - mi
            plsc.store_scatter(grp_p, [pos], hp, mask=m)
            return gpos + plsc.all_reduce_population_count(m)[0]
        return lax.fori_loop(0, nhit_vecs, body, gpos)

    gpos = jnp.int32(0)
    for gi in range(_NGROUP):
        gpos = bucket(gi, gpos)
        bounds.append(gpos)

    c_of_k = [k * _L + lanes for k in range(4)]

    def rc0_of(s):
        return pl.multiple_of(jnp.minimum(s * _SLOT_W, _LAST_BASE), 128)

    def fire(s, p):
        pltpu.async_copy(
            table_t_hbm.at[pl.ds(0, EMBEDDING_DIM), pl.ds(rc0_of(s), _SLOT_W)],
            block_v.at[p],
            sems[p],
        )

    def wait_block(p):
        pltpu.make_async_copy(
            table_t_hbm.at[pl.ds(0, EMBEDDING_DIM), pl.ds(0, _SLOT_W)],
            block_v.at[p],
            sems[p],
        ).wait()

    def compress_slot(s):
        base_rel = (s - s_lo) * _SLOT_W
        rc0_rel = jnp.minimum(s * _SLOT_W, _LAST_BASE) - lo
        gi = (s - s_lo) >> 3
        gs = jnp.int32(0)
        ge = jnp.int32(0)
        for i in range(_NGROUP):
            gs = jnp.where(gi == i, bounds[i], gs)
            ge = jnp.where(gi == i, bounds[i + 1], ge)
        u_lo = gs // _L
        u_hi = (ge + _L - 1) // _L

        def scan_hits(u, cnt):
            hp = grp_p[pl.ds(u * _L, _L)]
            rel = hp & 0x7FFF
            p = u * _L + lanes
            m = ((rel - base_rel).astype(jnp.uint32) < _SLOT_W) \
                & (p >= gs) & (p < ge)
            mi = m.astype(jnp.int32)
            pos = cnt + plsc.cumsum(mi) - mi
            packed = (rel - rc0_rel) | ((hp >> 15) << 10)
            plsc.store_scatter(list_a, [pos], packed, mask=m)
            return cnt + plsc.all_reduce_population_count(m)[0]

        return lax.fori_loop(u_lo, u_hi, scan_hits, jnp.int32(0))

    def process(blk, ns):
        def hit_group(h, carry2):
            spv = list_a[pl.ds(h * _L, _L)]
            lrv = spv & 0x3FF
            bv = spv >> 10
            for j in range(_L):
                @pl.when(h * _L + j < ns)
                def _fire(j=j, lrv=lrv, bv=bv):
                    cv = jnp.full((_L,), lrv[j], jnp.int32)
                    for k in range(4):
                        vals = plsc.load_gather(blk, [c_of_k[k], cv])
                        stage_v[pl.ds(j * EMBEDDING_DIM + k * _L, _L)] = vals
                    off = pl.multiple_of(bv[j] * EMBEDDING_DIM, 8)
                    pltpu.async_copy(
                        stage_v.at[pl.ds(j * EMBEDDING_DIM, EMBEDDING_DIM)],
                        out_hbm.at[pl.ds(off, EMBEDDING_DIM)],
                        sem_out,
                    )
            for j in range(_L):
                @pl.when(h * _L + j < ns)
                def _drain(j=j):
                    pltpu.make_async_copy(
                        out_hbm.at[pl.ds(0, EMBEDDING_DIM)],
                        stage_v.at[pl.ds(j * EMBEDDING_DIM, EMBEDDING_DIM)],
                        sem_out,
                    ).wait()
            return carry2

        lax.fori_loop(0, (ns + _L - 1) // _L, hit_group, jnp.int32(0))

    @pl.when(s_lo < s_hi)
    def _prime():
        fire(s_lo, 0)

    def do_slot(i, carry):
        par = (i - s_lo) % 2

        @pl.when(i + 1 < s_hi)
        def _next():
            @pl.when(par == 0)
            def _():
                fire(i + 1, 1)

            @pl.when(par == 1)
            def _():
                fire(i + 1, 0)

        ns = compress_slot(i)

        @pl.when(par == 0)
        def _p0():
            wait_block(0)
            process(block_v.at[0], ns)

        @pl.when(par == 1)
        def _p1():
            wait_block(1)
            process(block_v.at[1], ns)

        return carry

    lax.fori_loop(s_lo, s_hi, do_slot, jnp.int32(0))


def kernel(geohash_indices, embedding_table):
    idx = geohash_indices.astype(jnp.int32)
    out_flat = _sc_gather(idx, embedding_table.T)
    return out_flat.reshape(BATCH, EMBEDDING_DIM)


# speedup vs baseline: 1.1609x; 1.1609x over previous
import functools

import jax
import jax.numpy as jnp
from jax import lax
from jax.experimental import pallas as pl
from jax.experimental.pallas import tpu as pltpu
from jax.experimental.pallas import tpu_sc as plsc

BATCH = 16384
EMBEDDING_DIM = 64
NUM_ROWS = 1_000_000

_INFO = plsc.get_sparse_core_info()
_NC = _INFO.num_cores
_NS = _INFO.num_subcores
_NW = _NC * _NS
_L = 16

_SLOT_W = 512
_NSLOT = (NUM_ROWS + _SLOT_W - 1) // _SLOT_W
_LAST_BASE = 999552
_ICHUNK = 1024
_NICHUNK = BATCH // _ICHUNK
_IVEC = _ICHUNK // _L
_NGROUP = 8


@functools.partial(
    pl.kernel,
    mesh=plsc.VectorSubcoreMesh(core_axis_name="c", subcore_axis_name="s"),
    out_type=jax.ShapeDtypeStruct((BATCH * EMBEDDING_DIM,), jnp.float32),
    scratch_types=[
        pltpu.VMEM((BATCH,), jnp.int32),
        pltpu.VMEM((BATCH,), jnp.int32),
        pltpu.VMEM((BATCH,), jnp.int32),
        pltpu.VMEM((2, EMBEDDING_DIM, _SLOT_W), jnp.float32),
        pltpu.VMEM((_L * EMBEDDING_DIM,), jnp.float32),
        pltpu.SemaphoreType.DMA,
        pltpu.SemaphoreType.DMA,
        pltpu.SemaphoreType.DMA,
    ],
    compiler_params=pltpu.CompilerParams(
        use_tc_tiling_on_sc=True,
        disable_bounds_checks=True,
        needs_layout_passes=False,
    ),
)
def _sc_gather(idx_hbm, table_t_hbm, out_hbm, idx_v, list_a, grp_p,
               block_v, stage_v, sem_b0, sem_b1, sem_out):
    wid = lax.axis_index("s") * _NC + lax.axis_index("c")
    s_lo = wid * _NSLOT // _NW
    s_hi = (wid + 1) * _NSLOT // _NW
    lo = s_lo * _SLOT_W
    width = jnp.minimum(s_hi * _SLOT_W, NUM_ROWS) - lo
    lanes = lax.iota(jnp.int32, _L)
    sems = [sem_b0, sem_b1]

    pltpu.sync_copy(idx_hbm, idx_v)

    def scan_strip(v, cnt):
        rel = idx_v[pl.ds(v * _L, _L)] - lo
        m = rel.astype(jnp.uint32) < width.astype(jnp.uint32)
        mi = m.astype(jnp.int32)
        pos = cnt + plsc.cumsum(mi) - mi
        packed = rel + ((v * _L + lanes) << 15)
        plsc.store_scatter(list_a, [pos], packed, mask=m)
        return cnt + plsc.all_reduce_population_count(m)[0]

    nhit = lax.fori_loop(0, BATCH // _L, scan_strip, jnp.int32(0))
    nhit_vecs = (nhit + _L - 1) // _L

    c_of_k = [k * _L + lanes for k in range(4)]

    def rc0_of(s):
        return pl.multiple_of(jnp.minimum(s * _SLOT_W, _LAST_BASE), 128)

    def fire(s, p):
        pltpu.async_copy(
            table_t_hbm.at[pl.ds(0, EMBEDDING_DIM), pl.ds(rc0_of(s), _SLOT_W)],
            block_v.at[p],
            sems[p],
        )

    def wait_block(p):
        pltpu.make_async_copy(
            table_t_hbm.at[pl.ds(0, EMBEDDING_DIM), pl.ds(0, _SLOT_W)],
            block_v.at[p],
            sems[p],
        ).wait()

    def compress_slot(s):
        base_rel = (s - s_lo) * _SLOT_W
        rc0_rel = jnp.minimum(s * _SLOT_W, _LAST_BASE) - lo

        def scan_hits(u, cnt):
            hp = list_a[pl.ds(u * _L, _L)]
            rel = hp & 0x7FFF
            m = ((rel - base_rel).astype(jnp.uint32) < _SLOT_W) \
                & (u * _L + lanes < nhit)
            mi = m.astype(jnp.int32)
            pos = cnt + plsc.cumsum(mi) - mi
            packed = (rel - rc0_rel) | ((hp >> 15) << 10)
            plsc.store_scatter(grp_p, [pos], packed, mask=m)
            return cnt + plsc.all_reduce_population_count(m)[0]

        return lax.fori_loop(0, nhit_vecs, scan_hits, jnp.int32(0))

    def process(blk, ns):
        def hit_group(h, carry2):
            spv = grp_p[pl.ds(h * _L, _L)]
            lrv = spv & 0x3FF
            bv = spv >> 10
            for j in range(_L):
                @pl.when(h * _L + j < ns)
                def _fire(j=j, lrv=lrv, bv=bv):
                    cv = jnp.full((_L,), lrv[j], jnp.int32)
                    for k in range(4):
                        vals = plsc.load_gather(blk, [c_of_k[k], cv])
                        stage_v[pl.ds(j * EMBEDDING_DIM + k * _L, _L)] = vals
                    off = pl.multiple_of(bv[j] * EMBEDDING_DIM, 8)
                    pltpu.async_copy(
                        stage_v.at[pl.ds(j * EMBEDDING_DIM, EMBEDDING_DIM)],
                        out_hbm.at[pl.ds(off, EMBEDDING_DIM)],
                        sem_out,
                    )
            for j in range(_L):
                @pl.when(h * _L + j < ns)
                def _drain(j=j):
                    pltpu.make_async_copy(
                        out_hbm.at[pl.ds(0, EMBEDDING_DIM)],
                        stage_v.at[pl.ds(j * EMBEDDING_DIM, EMBEDDING_DIM)],
                        sem_out,
                    ).wait()
            return carry2

        lax.fori_loop(0, (ns + _L - 1) // _L, hit_group, jnp.int32(0))

    @pl.when(s_lo < s_hi)
    def _prime():
        fire(s_lo, 0)

    def do_slot(i, carry):
        par = (i - s_lo) % 2

        @pl.when(i + 1 < s_hi)
        def _next():
            @pl.when(par == 0)
            def _():
                fire(i + 1, 1)

            @pl.when(par == 1)
            def _():
                fire(i + 1, 0)

        ns = compress_slot(i)

        @pl.when(par == 0)
        def _p0():
            wait_block(0)
            process(block_v.at[0], ns)

        @pl.when(par == 1)
        def _p1():
            wait_block(1)
            process(block_v.at[1], ns)

        return carry

    lax.fori_loop(s_lo, s_hi, do_slot, jnp.int32(0))


def kernel(geohash_indices, embedding_table):
    idx = geohash_indices.astype(jnp.int32)
    out_flat = _sc_gather(idx, embedding_table.T)
    return out_flat.reshape(BATCH, EMBEDDING_DIM)
